# Initial kernel scaffold; baseline (speedup 1.0000x reference)
import jax
import jax.numpy as jnp
from jax.experimental import pallas as pl


def kernel(x, edge_index, batch, device):
    raise NotImplementedError("write your pallas kernel here")


# speedup vs baseline: 5.1270x; 5.1270x over previous
import functools

import jax
import jax.numpy as jnp
from jax import lax
from jax.experimental import pallas as pl
from jax.experimental.pallas import tpu as pltpu
from jax.experimental.pallas import tpu_sc as plsc

NC = 2
NS = 16
L = 16
NW = NC * NS


def _wid():
    return lax.axis_index("s") * NC + lax.axis_index("c")


def _mesh():
    return plsc.VectorSubcoreMesh(core_axis_name="c", subcore_axis_name="s")


_SC_PARAMS = pltpu.CompilerParams(use_tc_tiling_on_sc=False,
                                  needs_layout_passes=False)


def _make_sumsq(N, D, E, CH):
    EPW = E // NW
    NCH = EPW // CH

    def body(row2, col2, xh, out_h, idx_r, idx_c,
             buf_r0, buf_r1, buf_c0, buf_c1, out_v,
             sem_r0, sem_r1, sem_c0, sem_c1):
        wid = _wid()
        cbase = wid * NCH
        ebase = wid * EPW
        pltpu.sync_copy(row2.at[pl.ds(cbase, NCH)], idx_r)
        pltpu.sync_copy(col2.at[pl.ds(cbase, NCH)], idx_c)

        bufs_r = (buf_r0, buf_r1)
        bufs_c = (buf_c0, buf_c1)
        sems_r = (sem_r0, sem_r1)
        sems_c = (sem_c0, sem_c1)

        def fire(j, slot):
            pltpu.async_copy(xh.at[idx_r.at[j]], bufs_r[slot], sems_r[slot])
            pltpu.async_copy(xh.at[idx_c.at[j]], bufs_c[slot], sems_c[slot])

        def wait(slot):
            pltpu.make_async_copy(xh.at[idx_r.at[0]], bufs_r[slot],
                                  sems_r[slot]).wait()
            pltpu.make_async_copy(xh.at[idx_c.at[0]], bufs_c[slot],
                                  sems_c[slot]).wait()

        def compute(j, slot):
            br = bufs_r[slot]
            bc = bufs_c[slot]
            for g in range(CH // L):
                lanes = jnp.arange(L, dtype=jnp.int32) + g * L

                def kbody(k, acc):
                    kk = jnp.full((L,), k, dtype=jnp.int32)
                    vr = plsc.load_gather(br, [lanes, kk])
                    vc = plsc.load_gather(bc, [lanes, kk])
                    d = vr - vc
                    return acc + d * d

                acc = lax.fori_loop(0, D, kbody,
                                    jnp.zeros((L,), jnp.float32), unroll=8)
                out_v[pl.ds(j * CH + g * L, L)] = acc

        fire(0, 0)

        def outer(jj, carry):
            for b in range(2):
                j = 2 * jj + b
                fire(j + 1, 1 - b)
                wait(b)
                compute(j, b)
            return carry

        lax.fori_loop(0, (NCH - 1) // 2, outer, 0)
        wait(0)
        compute(NCH - 1, 0)

        pltpu.sync_copy(out_v, out_h.at[pl.ds(ebase, EPW)])

    kern = pl.kernel(
        body,
        out_type=jax.ShapeDtypeStruct((E,), jnp.float32),
        mesh=_mesh(),
        compiler_params=_SC_PARAMS,
        scratch_types=[
            pltpu.VMEM((NCH, CH), jnp.int32),
            pltpu.VMEM((NCH, CH), jnp.int32),
            pltpu.VMEM((CH, D), jnp.float32),
            pltpu.VMEM((CH, D), jnp.float32),
            pltpu.VMEM((CH, D), jnp.float32),
            pltpu.VMEM((CH, D), jnp.float32),
            pltpu.VMEM((EPW,), jnp.float32),
            pltpu.SemaphoreType.DMA,
            pltpu.SemaphoreType.DMA,
            pltpu.SemaphoreType.DMA,
            pltpu.SemaphoreType.DMA,
        ],
    )
    return kern


def _aff_body(s_ref, o_ref):
    s = s_ref[...]
    a = jnp.sqrt(s)
    p = 1.0 / (a + 0.01)
    o_ref[...] = jnp.log(p / (1.0 - p))


def _make_scatter(E, NPAD, CH):
    EPW = E // NW
    NCH = EPW // CH
    SLC = NPAD // NS

    def body(row2, aff_h, sums_h, cnts_h, idx_r, aff_v, ones_v, zero_v,
             sums_s, cnts_s):
        c = lax.axis_index("c")
        s = lax.axis_index("s")
        wid = s * NC + c
        cbase = wid * NCH
        ebase = wid * EPW
        pltpu.sync_copy(row2.at[pl.ds(cbase, NCH)], idx_r)
        pltpu.sync_copy(aff_h.at[pl.ds(ebase, EPW)], aff_v)

        for t in range(CH // L):
            ones_v[pl.ds(t * L, L)] = jnp.ones((L,), jnp.float32)
        for t in range(SLC // L):
            zero_v[pl.ds(t * L, L)] = jnp.zeros((L,), jnp.float32)

        pltpu.sync_copy(zero_v, sums_s.at[pl.ds(s * SLC, SLC)])
        pltpu.sync_copy(zero_v, cnts_s.at[pl.ds(s * SLC, SLC)])
        plsc.subcore_barrier()

        def sbody(j, carry):
            pltpu.sync_copy(aff_v.at[pl.ds(j * CH, CH)],
                            sums_s.at[idx_r.at[j]], add=True)
            pltpu.sync_copy(ones_v, cnts_s.at[idx_r.at[j]], add=True)
            return carry

        lax.fori_loop(0, NCH, sbody, 0)
        plsc.subcore_barrier()

        pltpu.sync_copy(sums_s.at[pl.ds(s * SLC, SLC)],
                        sums_h.at[c].at[pl.ds(s * SLC, SLC)])
        pltpu.sync_copy(cnts_s.at[pl.ds(s * SLC, SLC)],
                        cnts_h.at[c].at[pl.ds(s * SLC, SLC)])

    kern = pl.kernel(
        body,
        out_type=(jax.ShapeDtypeStruct((NC, NPAD), jnp.float32),
                  jax.ShapeDtypeStruct((NC, NPAD), jnp.float32)),
        mesh=_mesh(),
        compiler_params=_SC_PARAMS,
        scratch_types=[
            pltpu.VMEM((NCH, CH), jnp.int32),
            pltpu.VMEM((EPW,), jnp.float32),
            pltpu.VMEM((CH,), jnp.float32),
            pltpu.VMEM((SLC,), jnp.float32),
            pltpu.VMEM_SHARED((NPAD,), jnp.float32),
            pltpu.VMEM_SHARED((NPAD,), jnp.float32),
        ],
    )
    return kern


def _make_thresh(E, NPAD, CH):
    EPW = E // NW
    SLT = NPAD // NS

    def body(sums_h, cnts_h, row_h, col_h, out_h,
             part_v, mean_slice_v, mean_v, row_v, col_v, thr_v, mean_s):
        c = lax.axis_index("c")
        s = lax.axis_index("s")
        wid = s * NC + c
        ebase = wid * EPW
        nbase = s * SLT

        pltpu.sync_copy(sums_h.at[0].at[pl.ds(nbase, SLT)], part_v.at[0])
        pltpu.sync_copy(sums_h.at[1].at[pl.ds(nbase, SLT)], part_v.at[1])
        pltpu.sync_copy(cnts_h.at[0].at[pl.ds(nbase, SLT)], part_v.at[2])
        pltpu.sync_copy(cnts_h.at[1].at[pl.ds(nbase, SLT)], part_v.at[3])
        for t in range(SLT // L):
            sl = pl.ds(t * L, L)
            ssum = part_v[0, sl] + part_v[1, sl]
            cnt = part_v[2, sl] + part_v[3, sl]
            mean = jnp.where(cnt > 0.0, ssum / jnp.maximum(cnt, 1.0), 0.0)
            mean_slice_v[sl] = mean
        pltpu.sync_copy(mean_slice_v, mean_s.at[pl.ds(nbase, SLT)])
        plsc.subcore_barrier()
        pltpu.sync_copy(mean_s, mean_v)

        pltpu.sync_copy(row_h.at[pl.ds(ebase, EPW)], row_v)
        pltpu.sync_copy(col_h.at[pl.ds(ebase, EPW)], col_v)

        def gbody(t, carry):
            sl = pl.ds(t * L, L)
            ri = row_v[sl]
            ci = col_v[sl]
            mr = plsc.load_gather(mean_v, [ri])
            mc = plsc.load_gather(mean_v, [ci])
            thr_v[sl] = jnp.minimum(mr, mc) * 0.1
            return carry

        lax.fori_loop(0, EPW // L, gbody, 0, unroll=4)
        pltpu.sync_copy(thr_v, out_h.at[pl.ds(ebase, EPW)])

    kern = pl.kernel(
        body,
        out_type=jax.ShapeDtypeStruct((E,), jnp.float32),
        mesh=_mesh(),
        compiler_params=_SC_PARAMS,
        scratch_types=[
            pltpu.VMEM((4, SLT), jnp.float32),
            pltpu.VMEM((SLT,), jnp.float32),
            pltpu.VMEM((NPAD,), jnp.float32),
            pltpu.VMEM((EPW,), jnp.int32),
            pltpu.VMEM((EPW,), jnp.int32),
            pltpu.VMEM((EPW,), jnp.float32),
            pltpu.VMEM_SHARED((NPAD,), jnp.float32),
        ],
    )
    return kern


def kernel(x, edge_index, batch, device):
    N, D = x.shape
    E = edge_index.shape[1]
    CH = 80
    assert E % (NW * CH) == 0 and D % L == 0
    NPAD = ((N + NW * L - 1) // (NW * L)) * (NW * L)

    row = edge_index[0]
    col = edge_index[1]
    row2 = row.reshape(E // CH, CH)
    col2 = col.reshape(E // CH, CH)

    sumsq = _make_sumsq(N, D, E, CH)(row2, col2, x)

    aff2 = pl.pallas_call(
        _aff_body,
        out_shape=jax.ShapeDtypeStruct((E // 128, 128), jnp.float32),
    )(sumsq.reshape(E // 128, 128))
    aff = aff2.reshape(E)

    sums, cnts = _make_scatter(E, NPAD, CH)(row2, aff)
    thresh = _make_thresh(E, NPAD, CH)(sums, cnts, row, col)
    return (aff, thresh, 0.0)


# speedup vs baseline: 28.8853x; 5.6339x over previous
import functools

import jax
import jax.numpy as jnp
from jax import lax
from jax.experimental import pallas as pl
from jax.experimental.pallas import tpu as pltpu
from jax.experimental.pallas import tpu_sc as plsc

NC = 2
NS = 16
L = 16
NW = NC * NS


def _wid():
    return lax.axis_index("s") * NC + lax.axis_index("c")


def _mesh():
    return plsc.VectorSubcoreMesh(core_axis_name="c", subcore_axis_name="s")


_SC_PARAMS = pltpu.CompilerParams(use_tc_tiling_on_sc=False,
                                  needs_layout_passes=False)


def _make_sumsq(N, D, E, CH):
    EPW = E // NW
    NCH = EPW // CH

    def body(row2, col2, xh, out_h, idx_r, idx_c,
             buf_r0, buf_r1, buf_c0, buf_c1, out_v,
             sem_r0, sem_r1, sem_c0, sem_c1):
        wid = _wid()
        cbase = wid * NCH
        ebase = wid * EPW
        pltpu.sync_copy(row2.at[pl.ds(cbase, NCH)], idx_r)
        pltpu.sync_copy(col2.at[pl.ds(cbase, NCH)], idx_c)

        bufs_r = (buf_r0, buf_r1)
        bufs_c = (buf_c0, buf_c1)
        sems_r = (sem_r0, sem_r1)
        sems_c = (sem_c0, sem_c1)

        def fire(j, slot):
            pltpu.async_copy(xh.at[idx_r.at[j]], bufs_r[slot], sems_r[slot])
            pltpu.async_copy(xh.at[idx_c.at[j]], bufs_c[slot], sems_c[slot])

        def wait(slot):
            pltpu.make_async_copy(xh.at[idx_r.at[0]], bufs_r[slot],
                                  sems_r[slot]).wait()
            pltpu.make_async_copy(xh.at[idx_c.at[0]], bufs_c[slot],
                                  sems_c[slot]).wait()

        def compute(j, slot):
            br = bufs_r[slot]
            bc = bufs_c[slot]
            iot = jnp.arange(L, dtype=jnp.int32)
            for g in range(CH // L):
                lanes = iot + g * L

                def kbody(k, acc):
                    kk = (iot + k) & (D - 1)
                    vr = plsc.load_gather(br, [lanes, kk])
                    vc = plsc.load_gather(bc, [lanes, kk])
                    d = vr - vc
                    return acc + d * d

                acc = lax.fori_loop(0, D, kbody,
                                    jnp.zeros((L,), jnp.float32), unroll=8)
                out_v[pl.ds(j * CH + g * L, L)] = acc

        fire(0, 0)

        def outer(jj, carry):
            for b in range(2):
                j = 2 * jj + b
                fire(j + 1, 1 - b)
                wait(b)
                compute(j, b)
            return carry

        lax.fori_loop(0, (NCH - 1) // 2, outer, 0)
        wait(0)
        compute(NCH - 1, 0)

        pltpu.sync_copy(out_v, out_h.at[pl.ds(ebase, EPW)])

    kern = pl.kernel(
        body,
        out_type=jax.ShapeDtypeStruct((E,), jnp.float32),
        mesh=_mesh(),
        compiler_params=_SC_PARAMS,
        scratch_types=[
            pltpu.VMEM((NCH, CH), jnp.int32),
            pltpu.VMEM((NCH, CH), jnp.int32),
            pltpu.VMEM((CH, D), jnp.float32),
            pltpu.VMEM((CH, D), jnp.float32),
            pltpu.VMEM((CH, D), jnp.float32),
            pltpu.VMEM((CH, D), jnp.float32),
            pltpu.VMEM((EPW,), jnp.float32),
            pltpu.SemaphoreType.DMA,
            pltpu.SemaphoreType.DMA,
            pltpu.SemaphoreType.DMA,
            pltpu.SemaphoreType.DMA,
        ],
    )
    return kern


def _aff_body(s_ref, o_ref):
    s = s_ref[...]
    a = jnp.sqrt(s)
    p = 1.0 / (a + 0.01)
    o_ref[...] = jnp.log(p / (1.0 - p))


def _make_scatter(E, NPAD, CH):
    EPW = E // NW
    NCH = EPW // CH
    SLC = NPAD // NS

    def body(row2, aff_h, sums_h, cnts_h, idx_r, aff_v, ones_v, zero_v,
             sums_s, cnts_s):
        c = lax.axis_index("c")
        s = lax.axis_index("s")
        wid = s * NC + c
        cbase = wid * NCH
        ebase = wid * EPW
        pltpu.sync_copy(row2.at[pl.ds(cbase, NCH)], idx_r)
        pltpu.sync_copy(aff_h.at[pl.ds(ebase, EPW)], aff_v)

        for t in range(CH // L):
            ones_v[pl.ds(t * L, L)] = jnp.ones((L,), jnp.float32)
        for t in range(SLC // L):
            zero_v[pl.ds(t * L, L)] = jnp.zeros((L,), jnp.float32)

        pltpu.sync_copy(zero_v, sums_s.at[pl.ds(s * SLC, SLC)])
        pltpu.sync_copy(zero_v, cnts_s.at[pl.ds(s * SLC, SLC)])
        plsc.subcore_barrier()

        def sbody(j, carry):
            pltpu.sync_copy(aff_v.at[pl.ds(j * CH, CH)],
                            sums_s.at[idx_r.at[j]], add=True)
            pltpu.sync_copy(ones_v, cnts_s.at[idx_r.at[j]], add=True)
            return carry

        lax.fori_loop(0, NCH, sbody, 0)
        plsc.subcore_barrier()

        pltpu.sync_copy(sums_s.at[pl.ds(s * SLC, SLC)],
                        sums_h.at[c].at[pl.ds(s * SLC, SLC)])
        pltpu.sync_copy(cnts_s.at[pl.ds(s * SLC, SLC)],
                        cnts_h.at[c].at[pl.ds(s * SLC, SLC)])

    kern = pl.kernel(
        body,
        out_type=(jax.ShapeDtypeStruct((NC, NPAD), jnp.float32),
                  jax.ShapeDtypeStruct((NC, NPAD), jnp.float32)),
        mesh=_mesh(),
        compiler_params=_SC_PARAMS,
        scratch_types=[
            pltpu.VMEM((NCH, CH), jnp.int32),
            pltpu.VMEM((EPW,), jnp.float32),
            pltpu.VMEM((CH,), jnp.float32),
            pltpu.VMEM((SLC,), jnp.float32),
            pltpu.VMEM_SHARED((NPAD,), jnp.float32),
            pltpu.VMEM_SHARED((NPAD,), jnp.float32),
        ],
    )
    return kern


def _make_thresh(E, NPAD, CH):
    EPW = E // NW
    SLT = NPAD // NS

    def body(sums_h, cnts_h, row_h, col_h, out_h,
             part_v, mean_slice_v, mean_v, row_v, col_v, thr_v, mean_s):
        c = lax.axis_index("c")
        s = lax.axis_index("s")
        wid = s * NC + c
        ebase = wid * EPW
        nbase = s * SLT

        pltpu.sync_copy(sums_h.at[0].at[pl.ds(nbase, SLT)], part_v.at[0])
        pltpu.sync_copy(sums_h.at[1].at[pl.ds(nbase, SLT)], part_v.at[1])
        pltpu.sync_copy(cnts_h.at[0].at[pl.ds(nbase, SLT)], part_v.at[2])
        pltpu.sync_copy(cnts_h.at[1].at[pl.ds(nbase, SLT)], part_v.at[3])
        for t in range(SLT // L):
            sl = pl.ds(t * L, L)
            ssum = part_v[0, sl] + part_v[1, sl]
            cnt = part_v[2, sl] + part_v[3, sl]
            mean = jnp.where(cnt > 0.0, ssum / jnp.maximum(cnt, 1.0), 0.0)
            mean_slice_v[sl] = mean
        pltpu.sync_copy(mean_slice_v, mean_s.at[pl.ds(nbase, SLT)])
        plsc.subcore_barrier()
        pltpu.sync_copy(mean_s, mean_v)

        pltpu.sync_copy(row_h.at[pl.ds(ebase, EPW)], row_v)
        pltpu.sync_copy(col_h.at[pl.ds(ebase, EPW)], col_v)

        def gbody(t, carry):
            sl = pl.ds(t * L, L)
            ri = row_v[sl]
            ci = col_v[sl]
            mr = plsc.load_gather(mean_v, [ri])
            mc = plsc.load_gather(mean_v, [ci])
            thr_v[sl] = jnp.minimum(mr, mc) * 0.1
            return carry

        lax.fori_loop(0, EPW // L, gbody, 0, unroll=4)
        pltpu.sync_copy(thr_v, out_h.at[pl.ds(ebase, EPW)])

    kern = pl.kernel(
        body,
        out_type=jax.ShapeDtypeStruct((E,), jnp.float32),
        mesh=_mesh(),
        compiler_params=_SC_PARAMS,
        scratch_types=[
            pltpu.VMEM((4, SLT), jnp.float32),
            pltpu.VMEM((SLT,), jnp.float32),
            pltpu.VMEM((NPAD,), jnp.float32),
            pltpu.VMEM((EPW,), jnp.int32),
            pltpu.VMEM((EPW,), jnp.int32),
            pltpu.VMEM((EPW,), jnp.float32),
            pltpu.VMEM_SHARED((NPAD,), jnp.float32),
        ],
    )
    return kern


def kernel(x, edge_index, batch, device):
    N, D = x.shape
    E = edge_index.shape[1]
    CH = 80
    assert E % (NW * CH) == 0 and D % L == 0
    NPAD = ((N + NW * L - 1) // (NW * L)) * (NW * L)

    row = edge_index[0]
    col = edge_index[1]
    row2 = row.reshape(E // CH, CH)
    col2 = col.reshape(E // CH, CH)

    sumsq = _make_sumsq(N, D, E, CH)(row2, col2, x)

    aff2 = pl.pallas_call(
        _aff_body,
        out_shape=jax.ShapeDtypeStruct((E // 128, 128), jnp.float32),
    )(sumsq.reshape(E // 128, 128))
    aff = aff2.reshape(E)

    sums, cnts = _make_scatter(E, NPAD, CH)(row2, aff)
    thresh = _make_thresh(E, NPAD, CH)(sums, cnts, row, col)
    return (aff, thresh, 0.0)


# speedup vs baseline: 29.1409x; 1.0089x over previous
import functools

import jax
import jax.numpy as jnp
from jax import lax
from jax.experimental import pallas as pl
from jax.experimental.pallas import tpu as pltpu
from jax.experimental.pallas import tpu_sc as plsc

NC = 2
NS = 16
L = 16
NW = NC * NS


def _wid():
    return lax.axis_index("s") * NC + lax.axis_index("c")


def _mesh():
    return plsc.VectorSubcoreMesh(core_axis_name="c", subcore_axis_name="s")


_SC_PARAMS = pltpu.CompilerParams(use_tc_tiling_on_sc=False,
                                  needs_layout_passes=False)


def _make_sumsq(N, D, E, CH):
    EPW = E // NW
    NCH = EPW // CH
    W = D // 2

    def body(row2, col2, xh, out_h, idx_r, idx_c,
             buf_r0, buf_r1, buf_c0, buf_c1, out_v,
             sem_r0, sem_r1, sem_c0, sem_c1):
        wid = _wid()
        cbase = wid * NCH
        ebase = wid * EPW
        pltpu.sync_copy(row2.at[pl.ds(cbase, NCH)], idx_r)
        pltpu.sync_copy(col2.at[pl.ds(cbase, NCH)], idx_c)

        bufs_r = (buf_r0, buf_r1)
        bufs_c = (buf_c0, buf_c1)
        sems_r = (sem_r0, sem_r1)
        sems_c = (sem_c0, sem_c1)

        def fire(j, slot):
            pltpu.async_copy(xh.at[idx_r.at[j]], bufs_r[slot], sems_r[slot])
            pltpu.async_copy(xh.at[idx_c.at[j]], bufs_c[slot], sems_c[slot])

        def wait(slot):
            pltpu.make_async_copy(xh.at[idx_r.at[0]], bufs_r[slot],
                                  sems_r[slot]).wait()
            pltpu.make_async_copy(xh.at[idx_c.at[0]], bufs_c[slot],
                                  sems_c[slot]).wait()

        def compute(j, slot):
            br = bufs_r[slot]
            bc = bufs_c[slot]
            iot = jnp.arange(L, dtype=jnp.int32)
            himask = jnp.full((L,), -65536, dtype=jnp.int32)
            for g in range(CH // L):
                lanes = iot + g * L

                def kbody(k, acc):
                    kk = (iot + k) & (W - 1)
                    vr = plsc.load_gather(br, [lanes, kk])
                    vc = plsc.load_gather(bc, [lanes, kk])
                    rlo = plsc.bitcast(lax.shift_left(vr, 16), jnp.float32)
                    clo = plsc.bitcast(lax.shift_left(vc, 16), jnp.float32)
                    rhi = plsc.bitcast(vr & himask, jnp.float32)
                    chi = plsc.bitcast(vc & himask, jnp.float32)
                    dlo = rlo - clo
                    dhi = rhi - chi
                    return acc + dlo * dlo + dhi * dhi

                acc = lax.fori_loop(0, W, kbody,
                                    jnp.zeros((L,), jnp.float32), unroll=8)
                out_v[pl.ds(j * CH + g * L, L)] = acc

        fire(0, 0)

        def outer(jj, carry):
            for b in range(2):
                j = 2 * jj + b
                fire(j + 1, 1 - b)
                wait(b)
                compute(j, b)
            return carry

        lax.fori_loop(0, (NCH - 1) // 2, outer, 0)
        wait(0)
        compute(NCH - 1, 0)

        pltpu.sync_copy(out_v, out_h.at[pl.ds(ebase, EPW)])

    kern = pl.kernel(
        body,
        out_type=jax.ShapeDtypeStruct((E,), jnp.float32),
        mesh=_mesh(),
        compiler_params=_SC_PARAMS,
        scratch_types=[
            pltpu.VMEM((NCH, CH), jnp.int32),
            pltpu.VMEM((NCH, CH), jnp.int32),
            pltpu.VMEM((CH, W), jnp.int32),
            pltpu.VMEM((CH, W), jnp.int32),
            pltpu.VMEM((CH, W), jnp.int32),
            pltpu.VMEM((CH, W), jnp.int32),
            pltpu.VMEM((EPW,), jnp.float32),
            pltpu.SemaphoreType.DMA,
            pltpu.SemaphoreType.DMA,
            pltpu.SemaphoreType.DMA,
            pltpu.SemaphoreType.DMA,
        ],
    )
    return kern


def _aff_body(s_ref, o_ref):
    s = s_ref[...]
    a = jnp.sqrt(s)
    p = 1.0 / (a + 0.01)
    o_ref[...] = jnp.log(p / (1.0 - p))


def _make_scatter(E, NPAD, CH):
    EPW = E // NW
    NCH = EPW // CH
    SLC = NPAD // NS

    def body(row2, aff_h, sums_h, cnts_h, idx_r, aff_v, ones_v, zero_v,
             sums_s, cnts_s):
        c = lax.axis_index("c")
        s = lax.axis_index("s")
        wid = s * NC + c
        cbase = wid * NCH
        ebase = wid * EPW
        pltpu.sync_copy(row2.at[pl.ds(cbase, NCH)], idx_r)
        pltpu.sync_copy(aff_h.at[pl.ds(ebase, EPW)], aff_v)

        for t in range(CH // L):
            ones_v[pl.ds(t * L, L)] = jnp.ones((L,), jnp.float32)
        for t in range(SLC // L):
            zero_v[pl.ds(t * L, L)] = jnp.zeros((L,), jnp.float32)

        pltpu.sync_copy(zero_v, sums_s.at[pl.ds(s * SLC, SLC)])
        pltpu.sync_copy(zero_v, cnts_s.at[pl.ds(s * SLC, SLC)])
        plsc.subcore_barrier()

        def sbody(j, carry):
            pltpu.sync_copy(aff_v.at[pl.ds(j * CH, CH)],
                            sums_s.at[idx_r.at[j]], add=True)
            pltpu.sync_copy(ones_v, cnts_s.at[idx_r.at[j]], add=True)
            return carry

        lax.fori_loop(0, NCH, sbody, 0)
        plsc.subcore_barrier()

        pltpu.sync_copy(sums_s.at[pl.ds(s * SLC, SLC)],
                        sums_h.at[c].at[pl.ds(s * SLC, SLC)])
        pltpu.sync_copy(cnts_s.at[pl.ds(s * SLC, SLC)],
                        cnts_h.at[c].at[pl.ds(s * SLC, SLC)])

    kern = pl.kernel(
        body,
        out_type=(jax.ShapeDtypeStruct((NC, NPAD), jnp.float32),
                  jax.ShapeDtypeStruct((NC, NPAD), jnp.float32)),
        mesh=_mesh(),
        compiler_params=_SC_PARAMS,
        scratch_types=[
            pltpu.VMEM((NCH, CH), jnp.int32),
            pltpu.VMEM((EPW,), jnp.float32),
            pltpu.VMEM((CH,), jnp.float32),
            pltpu.VMEM((SLC,), jnp.float32),
            pltpu.VMEM_SHARED((NPAD,), jnp.float32),
            pltpu.VMEM_SHARED((NPAD,), jnp.float32),
        ],
    )
    return kern


def _make_thresh(E, NPAD, CH):
    EPW = E // NW
    SLT = NPAD // NS

    def body(sums_h, cnts_h, row_h, col_h, out_h,
             part_v, mean_slice_v, mean_v, row_v, col_v, thr_v, mean_s):
        c = lax.axis_index("c")
        s = lax.axis_index("s")
        wid = s * NC + c
        ebase = wid * EPW
        nbase = s * SLT

        pltpu.sync_copy(sums_h.at[0].at[pl.ds(nbase, SLT)], part_v.at[0])
        pltpu.sync_copy(sums_h.at[1].at[pl.ds(nbase, SLT)], part_v.at[1])
        pltpu.sync_copy(cnts_h.at[0].at[pl.ds(nbase, SLT)], part_v.at[2])
        pltpu.sync_copy(cnts_h.at[1].at[pl.ds(nbase, SLT)], part_v.at[3])
        for t in range(SLT // L):
            sl = pl.ds(t * L, L)
            ssum = part_v[0, sl] + part_v[1, sl]
            cnt = part_v[2, sl] + part_v[3, sl]
            mean = jnp.where(cnt > 0.0, ssum / jnp.maximum(cnt, 1.0), 0.0)
            mean_slice_v[sl] = mean
        pltpu.sync_copy(mean_slice_v, mean_s.at[pl.ds(nbase, SLT)])
        plsc.subcore_barrier()
        pltpu.sync_copy(mean_s, mean_v)

        pltpu.sync_copy(row_h.at[pl.ds(ebase, EPW)], row_v)
        pltpu.sync_copy(col_h.at[pl.ds(ebase, EPW)], col_v)

        def gbody(t, carry):
            sl = pl.ds(t * L, L)
            ri = row_v[sl]
            ci = col_v[sl]
            mr = plsc.load_gather(mean_v, [ri])
            mc = plsc.load_gather(mean_v, [ci])
            thr_v[sl] = jnp.minimum(mr, mc) * 0.1
            return carry

        lax.fori_loop(0, EPW // L, gbody, 0, unroll=4)
        pltpu.sync_copy(thr_v, out_h.at[pl.ds(ebase, EPW)])

    kern = pl.kernel(
        body,
        out_type=jax.ShapeDtypeStruct((E,), jnp.float32),
        mesh=_mesh(),
        compiler_params=_SC_PARAMS,
        scratch_types=[
            pltpu.VMEM((4, SLT), jnp.float32),
            pltpu.VMEM((SLT,), jnp.float32),
            pltpu.VMEM((NPAD,), jnp.float32),
            pltpu.VMEM((EPW,), jnp.int32),
            pltpu.VMEM((EPW,), jnp.int32),
            pltpu.VMEM((EPW,), jnp.float32),
            pltpu.VMEM_SHARED((NPAD,), jnp.float32),
        ],
    )
    return kern


def kernel(x, edge_index, batch, device):
    N, D = x.shape
    E = edge_index.shape[1]
    CH = 80
    assert E % (NW * CH) == 0 and D % L == 0
    NPAD = ((N + NW * L - 1) // (NW * L)) * (NW * L)

    row = edge_index[0]
    col = edge_index[1]
    row2 = row.reshape(E // CH, CH)
    col2 = col.reshape(E // CH, CH)

    xi = lax.bitcast_convert_type(
        x.astype(jnp.bfloat16).reshape(N, D // 2, 2), jnp.int32)

    sumsq = _make_sumsq(N, D, E, CH)(row2, col2, xi)

    aff2 = pl.pallas_call(
        _aff_body,
        out_shape=jax.ShapeDtypeStruct((E // 128, 128), jnp.float32),
    )(sumsq.reshape(E // 128, 128))
    aff = aff2.reshape(E)

    sums, cnts = _make_scatter(E, NPAD, CH)(row2, aff)
    thresh = _make_thresh(E, NPAD, CH)(sums, cnts, row, col)
    return (aff, thresh, 0.0)
